# Initial kernel scaffold; baseline (speedup 1.0000x reference)
import jax
import jax.numpy as jnp
from jax.experimental import pallas as pl


def kernel(x, table):
    raise NotImplementedError("write your pallas kernel here")


# speedup vs baseline: 1.8499x; 1.8499x over previous
import functools
import jax
import jax.numpy as jnp
from jax import lax
from jax.experimental import pallas as pl
from jax.experimental.pallas import tpu as pltpu, tpu_sc as plsc

_D = 64
_B = 16384 * 50


def _make_gather(B, D):
  info = plsc.get_sparse_core_info()
  NC, NS = info.num_cores, info.num_subcores
  NW = NC * NS
  assert B % NW == 0
  b_per_w = B // NW
  C = 800
  assert b_per_w % C == 0
  n_chunks = b_per_w // C

  mesh = plsc.VectorSubcoreMesh(core_axis_name="c", subcore_axis_name="s")

  @functools.partial(
      pl.kernel,
      out_type=jax.ShapeDtypeStruct((B, D), jnp.float32),
      mesh=mesh,
      compiler_params=pltpu.CompilerParams(use_tc_tiling_on_sc=False),
      scratch_types=[
          pltpu.VMEM((b_per_w,), jnp.int32),
          pltpu.VMEM((C, D), jnp.float32),
          pltpu.SemaphoreType.DMA,
      ],
  )
  def gather_kernel(idx_hbm, table_hbm, out_hbm, idx_v, rows_v, gsem):
    wid = lax.axis_index("s") * NC + lax.axis_index("c")
    base = wid * b_per_w
    pltpu.sync_copy(idx_hbm.at[pl.ds(base, b_per_w)], idx_v)

    @pl.loop(0, n_chunks)
    def _chunk(c):
      off = c * C
      pltpu.async_copy(
          table_hbm.at[idx_v.at[pl.ds(off, C)]], rows_v, gsem).wait()
      pltpu.sync_copy(rows_v, out_hbm.at[pl.ds(base + off, C)])

  return gather_kernel


_gather = _make_gather(_B, _D)


def kernel(x, table):
  idx = x.reshape(-1).astype(jnp.int32)
  out = _gather(idx, table)
  return out.reshape(x.shape + (table.shape[1],))


# speedup vs baseline: 1.8724x; 1.0122x over previous
import functools
import jax
import jax.numpy as jnp
from jax import lax
from jax.experimental import pallas as pl
from jax.experimental.pallas import tpu as pltpu, tpu_sc as plsc

_D = 64
_B = 16384 * 50


def _make_gather(B, D):
  info = plsc.get_sparse_core_info()
  NC, NS = info.num_cores, info.num_subcores
  NW = NC * NS
  assert B % NW == 0
  b_per_w = B // NW
  C = 400
  NBUF = 4
  assert b_per_w % (C * NBUF) == 0
  n_chunks = b_per_w // C

  mesh = plsc.VectorSubcoreMesh(core_axis_name="c", subcore_axis_name="s")

  @functools.partial(
      pl.kernel,
      out_type=jax.ShapeDtypeStruct((B, D), jnp.float32),
      mesh=mesh,
      compiler_params=pltpu.CompilerParams(use_tc_tiling_on_sc=False),
      scratch_types=[
          pltpu.VMEM((b_per_w,), jnp.int32),
          pltpu.VMEM((NBUF, C, D), jnp.float32),
          [pltpu.SemaphoreType.DMA] * NBUF,
          [pltpu.SemaphoreType.DMA] * NBUF,
      ],
  )
  def gather_kernel(idx_hbm, table_hbm, out_hbm, idx_v, rows_v, gsems, wsems):
    wid = lax.axis_index("s") * NC + lax.axis_index("c")
    base = wid * b_per_w
    pltpu.sync_copy(idx_hbm.at[pl.ds(base, b_per_w)], idx_v)

    def start_gather(c, b):
      pltpu.async_copy(
          table_hbm.at[idx_v.at[pl.ds(c * C, C)]], rows_v.at[b], gsems[b])

    def wait_gather(b):
      pltpu.make_async_copy(
          table_hbm.at[idx_v.at[pl.ds(0, C)]], rows_v.at[b], gsems[b]).wait()

    def start_write(c, b):
      pltpu.async_copy(
          rows_v.at[b], out_hbm.at[pl.ds(base + c * C, C)], wsems[b])

    def wait_write(b):
      pltpu.make_async_copy(
          rows_v.at[b], out_hbm.at[pl.ds(base, C)], wsems[b]).wait()

    for c in range(NBUF - 1):
      start_gather(c, c)

    for j in range(NBUF):
      wait_gather(j)
      start_write(j, j)
      pf = j + NBUF - 1
      if pf >= NBUF:
        wait_write(pf % NBUF)
      start_gather(pf, pf % NBUF)

    @pl.loop(1, n_chunks // NBUF - 1)
    def _group(i):
      c0 = i * NBUF
      for j in range(NBUF):
        c = c0 + j
        wait_gather(j)
        start_write(c, j)
        wait_write((j + NBUF - 1) % NBUF)
        start_gather(c + NBUF - 1, (j + NBUF - 1) % NBUF)

    c0 = n_chunks - NBUF
    for j in range(NBUF):
      c = c0 + j
      wait_gather(j)
      start_write(c, j)
      if c + NBUF - 1 < n_chunks:
        wait_write((j + NBUF - 1) % NBUF)
        start_gather(c + NBUF - 1, (j + NBUF - 1) % NBUF)
    for j in range(NBUF):
      wait_write(j)

  return gather_kernel


_gather = _make_gather(_B, _D)


def kernel(x, table):
  idx = x.reshape(-1).astype(jnp.int32)
  out = _gather(idx, table)
  return out.reshape(x.shape + (table.shape[1],))
